# Initial kernel scaffold; baseline (speedup 1.0000x reference)
import jax
import jax.numpy as jnp
from jax.experimental import pallas as pl


def kernel(x, pos_y, batch_x, batch_y, W, b, ref_pos):
    raise NotImplementedError("write your pallas kernel here")


# speedup vs baseline: 11.1921x; 11.1921x over previous
import functools

import jax
import jax.numpy as jnp
from jax import lax
from jax.experimental import pallas as pl
from jax.experimental.pallas import tpu as pltpu
from jax.experimental.pallas import tpu_sc as plsc

IN_SIZE = 1024
OUT_REF = 2048
D_FEAT = 512
N_Y = 8192
POS_DIM = 3
KNN = 4

NC = 2
NS = 16
NW = NC * NS
NODES_PER_W = N_Y // NW
GRP = 16
NGRP = NODES_PER_W // GRP
ROWS_PER_GRP = GRP * KNN


def _z_body(w_ref, x_ref, b_ref, mx_ref, z_ref):
    xm = x_ref[...] * mx_ref[...]
    z_ref[...] = (
        jnp.dot(w_ref[...], xm, preferred_element_type=jnp.float32)
        + b_ref[...]
    )


def _compute_z(W, x, b2, mxf):
    bm = 256
    return pl.pallas_call(
        _z_body,
        grid=(OUT_REF // bm,),
        in_specs=[
            pl.BlockSpec((bm, IN_SIZE), lambda i: (i, 0)),
            pl.BlockSpec((IN_SIZE, D_FEAT), lambda i: (0, 0)),
            pl.BlockSpec((bm, 1), lambda i: (i, 0)),
            pl.BlockSpec((IN_SIZE, 1), lambda i: (0, 0)),
        ],
        out_specs=pl.BlockSpec((bm, D_FEAT), lambda i: (i, 0)),
        out_shape=jax.ShapeDtypeStruct((OUT_REF, D_FEAT), jnp.float32),
    )(W, x, b2, mxf)


def _topk_body(pos_ref, rpt_ref, myf_ref, topi_ref, wgt_ref):
    p = pos_ref[...]
    rt = rpt_ref[...]
    bm = p.shape[0]
    pn = jnp.sum(p * p, axis=1, keepdims=True)
    rn = jnp.sum(rt * rt, axis=0, keepdims=True)
    cross = jnp.dot(p, rt)
    d2 = pn + rn - 2.0 * cross
    iota = lax.broadcasted_iota(jnp.int32, (bm, OUT_REF), 1)
    BIG = jnp.float32(3.0e38)
    vals = []
    idxs = []
    for _ in range(KNN):
        v = jnp.min(d2, axis=1, keepdims=True)
        eq = d2 <= v
        idx = jnp.min(jnp.where(eq, iota, OUT_REF), axis=1, keepdims=True)
        d2 = jnp.where(iota == idx, BIG, d2)
        vals.append(v)
        idxs.append(idx)
    topv = jnp.concatenate(vals, axis=1)
    topi = jnp.concatenate(idxs, axis=1)
    dist = jnp.sqrt(jnp.maximum(topv, 0.0))
    w = 1.0 / (dist + 1e-8)
    w = w / jnp.sum(w, axis=1, keepdims=True)
    topi_ref[...] = topi
    wgt_ref[...] = w * myf_ref[...]


def _compute_topk(pos_y, ref_pos_t, myf):
    bm = 1024
    return pl.pallas_call(
        _topk_body,
        grid=(N_Y // bm,),
        in_specs=[
            pl.BlockSpec((bm, POS_DIM), lambda i: (i, 0)),
            pl.BlockSpec((POS_DIM, OUT_REF), lambda i: (0, 0)),
            pl.BlockSpec((bm, 1), lambda i: (i, 0)),
        ],
        out_specs=[
            pl.BlockSpec((bm, KNN), lambda i: (i, 0)),
            pl.BlockSpec((bm, KNN), lambda i: (i, 0)),
        ],
        out_shape=[
            jax.ShapeDtypeStruct((N_Y, KNN), jnp.int32),
            jax.ShapeDtypeStruct((N_Y, KNN), jnp.float32),
        ],
    )(pos_y, ref_pos_t, myf)


_GATHER_DNUMS = lax.GatherDimensionNumbers(
    offset_dims=(), collapsed_slice_dims=(0,), start_index_map=(0,))


def _lane_bcast(vec, lane):
    idx = jnp.full((16, 1), lane, dtype=jnp.int32)
    return lax.gather(vec, idx, _GATHER_DNUMS, slice_sizes=(1,),
                      mode=lax.GatherScatterMode.PROMISE_IN_BOUNDS)


def _sc_gather_body(z_hbm, topi_hbm, wgt_hbm, out_hbm,
                    idx_v, wgt_v, rows_v, out_v, sem):
    wid = lax.axis_index("s") * NC + lax.axis_index("c")

    def group(g, _):
        row = wid * NGRP + g
        pltpu.sync_copy(topi_hbm.at[row], idx_v)
        pltpu.sync_copy(wgt_hbm.at[row], wgt_v)
        pltpu.async_copy(z_hbm.at[idx_v], rows_v, sem).wait()
        for i in range(GRP):
            wv = wgt_v[pl.ds((i // 4) * 16, 16)]
            wb = [_lane_bcast(wv, 4 * (i % 4) + k) for k in range(KNN)]

            def chunk(c, _):
                sl = pl.ds(c * 16, 16)
                acc = wb[0] * rows_v[KNN * i, sl]
                for k in range(1, KNN):
                    acc = acc + wb[k] * rows_v[KNN * i + k, sl]
                out_v[i, sl] = acc
                return _

            lax.fori_loop(0, D_FEAT // 16, chunk, None, unroll=4)
        pltpu.sync_copy(out_v, out_hbm.at[pl.ds((wid * NGRP + g) * GRP, GRP)])
        return _

    lax.fori_loop(0, NGRP, group, None)


def _sc_gather(Z, topi_rows, wgt_rows):
    mesh = plsc.VectorSubcoreMesh(core_axis_name="c", subcore_axis_name="s")
    return pl.kernel(
        _sc_gather_body,
        out_type=jax.ShapeDtypeStruct((N_Y, D_FEAT), jnp.float32),
        mesh=mesh,
        scratch_types=[
            pltpu.VMEM((ROWS_PER_GRP,), jnp.int32),
            pltpu.VMEM((ROWS_PER_GRP,), jnp.float32),
            pltpu.VMEM((ROWS_PER_GRP, D_FEAT), jnp.float32),
            pltpu.VMEM((GRP, D_FEAT), jnp.float32),
            pltpu.SemaphoreType.DMA,
        ],
    )(Z, topi_rows, wgt_rows)


def kernel(x, pos_y, batch_x, batch_y, W, b, ref_pos):
    lab = batch_x[0]
    mxf = (batch_x == lab).astype(jnp.float32)[:, None]
    myf = (batch_y == lab).astype(jnp.float32)[:, None]
    Z = _compute_z(W, x, b[:, None], mxf)
    topi, wgt = _compute_topk(pos_y, ref_pos.T, myf)
    topi_rows = topi.reshape(NW * NGRP, ROWS_PER_GRP)
    wgt_rows = wgt.reshape(NW * NGRP, ROWS_PER_GRP)
    return _sc_gather(Z, topi_rows, wgt_rows)


# speedup vs baseline: 13.4256x; 1.1996x over previous
import functools

import jax
import jax.numpy as jnp
from jax import lax
from jax.experimental import pallas as pl
from jax.experimental.pallas import tpu as pltpu
from jax.experimental.pallas import tpu_sc as plsc

IN_SIZE = 1024
OUT_REF = 2048
D_FEAT = 512
N_Y = 8192
POS_DIM = 3
KNN = 4

NC = 2
NS = 16
NW = NC * NS
NODES_PER_W = N_Y // NW
GRP = 16
NGRP = NODES_PER_W // GRP
ROWS_PER_GRP = GRP * KNN


def _z_body(w_ref, x_ref, b_ref, mx_ref, z_ref):
    xm = x_ref[...] * mx_ref[...]
    z_ref[...] = (
        jnp.dot(w_ref[...], xm, preferred_element_type=jnp.float32)
        + b_ref[...]
    )


def _compute_z(W, x, b2, mxf):
    bm = 256
    return pl.pallas_call(
        _z_body,
        grid=(OUT_REF // bm,),
        in_specs=[
            pl.BlockSpec((bm, IN_SIZE), lambda i: (i, 0)),
            pl.BlockSpec((IN_SIZE, D_FEAT), lambda i: (0, 0)),
            pl.BlockSpec((bm, 1), lambda i: (i, 0)),
            pl.BlockSpec((IN_SIZE, 1), lambda i: (0, 0)),
        ],
        out_specs=pl.BlockSpec((bm, D_FEAT), lambda i: (i, 0)),
        out_shape=jax.ShapeDtypeStruct((OUT_REF, D_FEAT), jnp.float32),
    )(W, x, b2, mxf)


def _topk_body(pos_ref, rpt_ref, myf_ref, topi_ref, wgt_ref):
    p = pos_ref[...]
    rt = rpt_ref[...]
    bm = p.shape[0]
    pn = jnp.sum(p * p, axis=1, keepdims=True)
    rn = jnp.sum(rt * rt, axis=0, keepdims=True)
    cross = jnp.dot(p, rt)
    d2 = pn + rn - 2.0 * cross
    iota_f = lax.broadcasted_iota(jnp.int32, (bm, OUT_REF), 1).astype(jnp.float32)
    BIG = jnp.float32(3.0e38)
    BIGI = jnp.float32(4096.0)
    vals = []
    idxs = []
    for _ in range(KNN):
        v = jnp.min(d2, axis=1, keepdims=True)
        sel = jnp.where(d2 <= v, iota_f, BIGI)
        idxf = jnp.min(sel, axis=1, keepdims=True)
        d2 = jnp.where(sel == idxf, BIG, d2)
        vals.append(v)
        idxs.append(idxf)
    topv = jnp.concatenate(vals, axis=1)
    topi = jnp.concatenate(idxs, axis=1).astype(jnp.int32)
    dist = jnp.sqrt(jnp.maximum(topv, 0.0))
    w = 1.0 / (dist + 1e-8)
    w = w / jnp.sum(w, axis=1, keepdims=True)
    topi_ref[...] = topi
    wgt_ref[...] = w * myf_ref[...]


def _compute_topk(pos_y, ref_pos_t, myf):
    bm = 1024
    return pl.pallas_call(
        _topk_body,
        grid=(N_Y // bm,),
        in_specs=[
            pl.BlockSpec((bm, POS_DIM), lambda i: (i, 0)),
            pl.BlockSpec((POS_DIM, OUT_REF), lambda i: (0, 0)),
            pl.BlockSpec((bm, 1), lambda i: (i, 0)),
        ],
        out_specs=[
            pl.BlockSpec((bm, KNN), lambda i: (i, 0)),
            pl.BlockSpec((bm, KNN), lambda i: (i, 0)),
        ],
        out_shape=[
            jax.ShapeDtypeStruct((N_Y, KNN), jnp.int32),
            jax.ShapeDtypeStruct((N_Y, KNN), jnp.float32),
        ],
    )(pos_y, ref_pos_t, myf)


_GATHER_DNUMS = lax.GatherDimensionNumbers(
    offset_dims=(), collapsed_slice_dims=(0,), start_index_map=(0,))


def _lane_bcast(vec, lane):
    idx = jnp.full((16, 1), lane, dtype=jnp.int32)
    return lax.gather(vec, idx, _GATHER_DNUMS, slice_sizes=(1,),
                      mode=lax.GatherScatterMode.PROMISE_IN_BOUNDS)


def _sc_gather_body(z_hbm, topi_hbm, wgt_hbm, out_hbm,
                    idx_all, wgt_all, rows_v, out_v,
                    gsem0, gsem1, osem0, osem1):
    wid = lax.axis_index("s") * NC + lax.axis_index("c")
    row0 = wid * NGRP
    gsems = (gsem0, gsem1)
    osems = (osem0, osem1)

    pltpu.sync_copy(topi_hbm.at[pl.ds(row0, NGRP)], idx_all)
    pltpu.sync_copy(wgt_hbm.at[pl.ds(row0, NGRP)], wgt_all)

    def start_gather(g, p):
        pltpu.async_copy(z_hbm.at[idx_all.at[g]], rows_v.at[p], gsems[p])

    def wait_gather(g, p):
        pltpu.make_async_copy(z_hbm.at[idx_all.at[g]], rows_v.at[p],
                              gsems[p]).wait()

    def out_slice(g):
        return out_hbm.at[pl.ds((row0 + g) * GRP, GRP)]

    start_gather(0, 0)
    start_gather(1, 1)
    NSUP = NGRP // 2

    def superstep(s, carry):
        for p in range(2):
            g = 2 * s + p

            @pl.when(s > 0)
            def _():
                pltpu.make_async_copy(out_v.at[p], out_slice(g - 2),
                                      osems[p]).wait()

            wait_gather(g, p)
            for j in range(GRP // 4):
                wv = wgt_all[g, pl.ds(j * 16, 16)]
                for ii in range(4):
                    i = 4 * j + ii
                    wb = [_lane_bcast(wv, 4 * ii + k) for k in range(KNN)]

                    def chunk(c, _, i=i, wb=wb, p=p):
                        sl = pl.ds(c * 16, 16)
                        acc = wb[0] * rows_v[p, KNN * i, sl]
                        for k in range(1, KNN):
                            acc = acc + wb[k] * rows_v[p, KNN * i + k, sl]
                        out_v[p, i, sl] = acc
                        return _

                    lax.fori_loop(0, D_FEAT // 16, chunk, None, unroll=4)

            @pl.when(s < NSUP - 1)
            def _():
                start_gather(g + 2, p)

            pltpu.async_copy(out_v.at[p], out_slice(g), osems[p])
        return carry

    lax.fori_loop(0, NSUP, superstep, None)
    for p in range(2):
        pltpu.make_async_copy(out_v.at[p], out_slice(NGRP - 2 + p),
                              osems[p]).wait()


def _sc_gather(Z, topi_rows, wgt_rows):
    mesh = plsc.VectorSubcoreMesh(core_axis_name="c", subcore_axis_name="s")
    return pl.kernel(
        _sc_gather_body,
        out_type=jax.ShapeDtypeStruct((N_Y, D_FEAT), jnp.float32),
        mesh=mesh,
        scratch_types=[
            pltpu.VMEM((NGRP, ROWS_PER_GRP), jnp.int32),
            pltpu.VMEM((NGRP, ROWS_PER_GRP), jnp.float32),
            pltpu.VMEM((2, ROWS_PER_GRP, D_FEAT), jnp.float32),
            pltpu.VMEM((2, GRP, D_FEAT), jnp.float32),
            pltpu.SemaphoreType.DMA,
            pltpu.SemaphoreType.DMA,
            pltpu.SemaphoreType.DMA,
            pltpu.SemaphoreType.DMA,
        ],
    )(Z, topi_rows, wgt_rows)


def kernel(x, pos_y, batch_x, batch_y, W, b, ref_pos):
    lab = batch_x[0]
    mxf = (batch_x == lab).astype(jnp.float32)[:, None]
    myf = (batch_y == lab).astype(jnp.float32)[:, None]
    Z = _compute_z(W, x, b[:, None], mxf)
    topi, wgt = _compute_topk(pos_y, ref_pos.T, myf)
    topi_rows = topi.reshape(NW * NGRP, ROWS_PER_GRP)
    wgt_rows = wgt.reshape(NW * NGRP, ROWS_PER_GRP)
    return _sc_gather(Z, topi_rows, wgt_rows)


# speedup vs baseline: 18.8018x; 1.4004x over previous
import functools

import jax
import jax.numpy as jnp
from jax import lax
from jax.experimental import pallas as pl
from jax.experimental.pallas import tpu as pltpu
from jax.experimental.pallas import tpu_sc as plsc

IN_SIZE = 1024
OUT_REF = 2048
D_FEAT = 512
N_Y = 8192
POS_DIM = 3
KNN = 4

NC = 2
NS = 16
NW = NC * NS
NODES_PER_W = N_Y // NW
GRP = 8
NGRP = NODES_PER_W // GRP
ROWS_PER_GRP = GRP * KNN
NBUF = 4


def _z_body(w_ref, x_ref, b_ref, bx_ref, z_ref):
    bx = bx_ref[...]
    mxf = (bx == bx[0:1, 0:1]).astype(jnp.float32)
    xm = x_ref[...] * mxf
    z_ref[...] = (
        jnp.dot(w_ref[...], xm, preferred_element_type=jnp.float32)
        + b_ref[...]
    )


def _compute_z(W, x, b2, bx2):
    bm = 256
    return pl.pallas_call(
        _z_body,
        grid=(OUT_REF // bm,),
        in_specs=[
            pl.BlockSpec((bm, IN_SIZE), lambda i: (i, 0)),
            pl.BlockSpec((IN_SIZE, D_FEAT), lambda i: (0, 0)),
            pl.BlockSpec((bm, 1), lambda i: (i, 0)),
            pl.BlockSpec((IN_SIZE, 1), lambda i: (0, 0)),
        ],
        out_specs=pl.BlockSpec((bm, D_FEAT), lambda i: (i, 0)),
        out_shape=jax.ShapeDtypeStruct((OUT_REF, D_FEAT), jnp.float32),
    )(W, x, b2, bx2)


def _topk_body(pos_ref, r_ref, by_ref, bx_ref, topi_ref, wgt_ref):
    p = pos_ref[...]
    r = r_ref[...]
    bm = p.shape[0]
    myf = (by_ref[...] == bx_ref[0:1, 0:1]).astype(jnp.float32)
    pn = jnp.sum(p * p, axis=1, keepdims=True)
    rn = jnp.sum(r * r, axis=1)[None, :]
    cross = lax.dot_general(p, r, (((1,), (1,)), ((), ())))
    d2 = pn + rn - 2.0 * cross
    iota_f = lax.broadcasted_iota(jnp.int32, (bm, OUT_REF), 1).astype(jnp.float32)
    BIG = jnp.float32(3.0e38)
    BIGI = jnp.float32(4096.0)
    vals = []
    idxs = []
    for kk in range(KNN):
        v = jnp.min(d2, axis=1, keepdims=True)
        sel = jnp.where(d2 <= v, iota_f, BIGI)
        idxf = jnp.min(sel, axis=1, keepdims=True)
        if kk < KNN - 1:
            d2 = jnp.where(sel == idxf, BIG, d2)
        vals.append(v)
        idxs.append(idxf)
    topv = jnp.concatenate(vals, axis=1)
    topi = jnp.concatenate(idxs, axis=1).astype(jnp.int32)
    dist = jnp.sqrt(jnp.maximum(topv, 0.0))
    w = 1.0 / (dist + 1e-8)
    w = w / jnp.sum(w, axis=1, keepdims=True)
    topi_ref[...] = topi
    wgt_ref[...] = w * myf


def _compute_topk(pos_y, ref_pos, by2, bx2, goff, gcnt):
    bm = 1024
    return pl.pallas_call(
        _topk_body,
        grid=(gcnt,),
        in_specs=[
            pl.BlockSpec((bm, POS_DIM), lambda i: (i + goff, 0)),
            pl.BlockSpec((OUT_REF, POS_DIM), lambda i: (0, 0)),
            pl.BlockSpec((bm, 1), lambda i: (i + goff, 0)),
            pl.BlockSpec((IN_SIZE, 1), lambda i: (0, 0)),
        ],
        out_specs=[
            pl.BlockSpec((bm, KNN), lambda i: (i, 0)),
            pl.BlockSpec((bm, KNN), lambda i: (i, 0)),
        ],
        out_shape=[
            jax.ShapeDtypeStruct((gcnt * bm, KNN), jnp.int32),
            jax.ShapeDtypeStruct((gcnt * bm, KNN), jnp.float32),
        ],
    )(pos_y, ref_pos, by2, bx2)


_GATHER_DNUMS = lax.GatherDimensionNumbers(
    offset_dims=(), collapsed_slice_dims=(0,), start_index_map=(0,))


def _lane_bcast(vec, lane):
    idx = jnp.full((16, 1), lane, dtype=jnp.int32)
    return lax.gather(vec, idx, _GATHER_DNUMS, slice_sizes=(1,),
                      mode=lax.GatherScatterMode.PROMISE_IN_BOUNDS)


def _sc_gather_body(z_hbm, topi_hbm, wgt_hbm, out_hbm,
                    idx_all, wgt_all, rows_v, out_v,
                    gsem0, gsem1, gsem2, gsem3, osem0, osem1,
                    NGRP, ROW_BASE):
    wid = lax.axis_index("s") * NC + lax.axis_index("c")
    row0 = wid * NGRP
    gsems = (gsem0, gsem1, gsem2, gsem3)
    osems = (osem0, osem1)

    pltpu.sync_copy(topi_hbm.at[pl.ds(row0, NGRP)], idx_all)
    pltpu.sync_copy(wgt_hbm.at[pl.ds(row0, NGRP)], wgt_all)

    def start_gather(g, p):
        pltpu.async_copy(z_hbm.at[idx_all.at[g]], rows_v.at[p], gsems[p])

    def wait_gather(g, p):
        pltpu.make_async_copy(z_hbm.at[idx_all.at[g]], rows_v.at[p],
                              gsems[p]).wait()

    def out_slice(g):
        return out_hbm.at[pl.ds((ROW_BASE + row0 + g) * GRP, GRP)]

    for g0 in range(NBUF - 1):
        start_gather(g0, g0)

    def superstep(s, carry):
        for p4 in range(NBUF):
            g = NBUF * s + p4
            slot = p4
            q = p4 % 2

            @pl.when(g + NBUF - 1 < NGRP)
            def _():
                start_gather(g + NBUF - 1, (p4 + NBUF - 1) % NBUF)

            @pl.when(g >= 2)
            def _():
                pltpu.make_async_copy(out_v.at[q], out_slice(g - 2),
                                      osems[q]).wait()

            wait_gather(g, slot)
            for j in range(GRP // 4):
                wv = wgt_all[g, pl.ds(j * 16, 16)]
                for ii in range(4):
                    i = 4 * j + ii
                    wb = [_lane_bcast(wv, 4 * ii + k) for k in range(KNN)]

                    @plsc.parallel_loop(0, D_FEAT // 16, unroll=4)
                    def _(c, i=i, wb=wb, slot=slot, q=q):
                        sl = pl.ds(c * 16, 16)
                        acc = wb[0] * rows_v[slot, KNN * i, sl]
                        for k in range(1, KNN):
                            acc = acc + wb[k] * rows_v[slot, KNN * i + k, sl]
                        out_v[q, i, sl] = acc

            pltpu.async_copy(out_v.at[q], out_slice(g), osems[q])
        return carry

    lax.fori_loop(0, NGRP // NBUF, superstep, None)
    for q in range(2):
        pltpu.make_async_copy(out_v.at[q], out_slice(NGRP - 2 + q),
                              osems[q]).wait()


def _sc_gather(Zw, topi_rows, wgt_rows, n_y, row_base, out_ref):
    ngrp = n_y // (NW * GRP)
    body = functools.partial(_sc_gather_body, NGRP=ngrp, ROW_BASE=row_base)
    mesh = plsc.VectorSubcoreMesh(core_axis_name="c", subcore_axis_name="s")
    return pl.kernel(
        body,
        out_type=(),
        mesh=mesh,
        scratch_types=[
            pltpu.VMEM((ngrp, ROWS_PER_GRP), jnp.int32),
            pltpu.VMEM((ngrp, ROWS_PER_GRP), jnp.float32),
            pltpu.VMEM((NBUF, ROWS_PER_GRP, D_FEAT), jnp.float32),
            pltpu.VMEM((2, GRP, D_FEAT), jnp.float32),
            pltpu.SemaphoreType.DMA,
            pltpu.SemaphoreType.DMA,
            pltpu.SemaphoreType.DMA,
            pltpu.SemaphoreType.DMA,
            pltpu.SemaphoreType.DMA,
            pltpu.SemaphoreType.DMA,
        ],
    )(Zw, topi_rows, wgt_rows, out_ref)


NSPLIT = 2


def kernel(x, pos_y, batch_x, batch_y, W, b, ref_pos):
    bx2 = batch_x[:, None]
    by2 = batch_y[:, None]
    Z = _compute_z(W, x, b[:, None], bx2)
    h = N_Y // NSPLIT
    gcnt = h // 1024
    out_ref = jax.new_ref(jnp.zeros((N_Y, D_FEAT), jnp.float32))
    for s in range(NSPLIT):
        topi, wgt = _compute_topk(pos_y, ref_pos, by2, bx2, s * gcnt, gcnt)
        topi_rows = topi.reshape(h // GRP, ROWS_PER_GRP)
        wgt_rows = wgt.reshape(h // GRP, ROWS_PER_GRP)
        _sc_gather(Z, topi_rows, wgt_rows, h, s * (h // GRP), out_ref)
    return out_ref[...]
